# baseline (speedup 1.0000x reference)
import functools

import jax
import jax.numpy as jnp
from jax import lax
from jax.experimental import pallas as pl
from jax.experimental.pallas import tpu as pltpu
from jax.experimental.pallas import tpu_sc as plsc

VOCAB = 1000000
EMB = 32
BATCH = 4096
SEQ = 200
N = BATCH * SEQ
NW = 32
PER_W = N // NW
CHUNK = 128
G = PER_W // CHUNK
RING = 10
AHEAD = 5
L = 16


def _make_kernel():
    mesh = plsc.VectorSubcoreMesh(core_axis_name="c", subcore_axis_name="s")

    @functools.partial(
        pl.kernel,
        out_type=(
            jax.ShapeDtypeStruct((NW, G, CHUNK, EMB), jnp.float32),
            jax.ShapeDtypeStruct((NW, G, CHUNK), jnp.float32),
        ),
        mesh=mesh,
        compiler_params=pltpu.CompilerParams(use_tc_tiling_on_sc=False),
        scratch_types=(
            [
                pltpu.VMEM((G, CHUNK), jnp.int32),
                pltpu.VMEM((RING, CHUNK, EMB), jnp.float32),
                pltpu.VMEM((G, CHUNK), jnp.float32),
            ]
            + [pltpu.SemaphoreType.DMA] * RING
            + [pltpu.SemaphoreType.DMA] * RING
        ),
    )
    def emb_kernel(x_hbm, w_hbm, out_hbm, mask_hbm, idx_v, rows_v, mask_v,
                   *sems):
        gsems = sems[:RING]
        osems = sems[RING:]
        wid = lax.axis_index("s") * 2 + lax.axis_index("c")

        pltpu.sync_copy(x_hbm.at[wid], idx_v)

        def start_gather(g, r):
            pltpu.async_copy(w_hbm.at[idx_v.at[g]], rows_v.at[r], gsems[r])

        def wait_gather(g, r):
            pltpu.make_async_copy(
                w_hbm.at[idx_v.at[g]], rows_v.at[r], gsems[r]).wait()

        def start_out(g, r):
            pltpu.async_copy(rows_v.at[r], out_hbm.at[wid, g], osems[r])

        def wait_out(g, r):
            pltpu.make_async_copy(
                rows_v.at[r], out_hbm.at[wid, g], osems[r]).wait()

        def compute_mask(g):
            for j in range(CHUNK // L):
                v = idx_v[g, pl.ds(j * L, L)]
                mask_v[g, pl.ds(j * L, L)] = jnp.where(
                    v != 0, jnp.float32(1.0), jnp.float32(0.0))

        def visit(g, r, first_lap):
            compute_mask(g)
            wait_gather(g, r)
            start_out(g, r)
            g2 = g + AHEAD
            r2 = (r + AHEAD) % RING
            if first_lap:
                if g2 < RING:
                    start_gather(g2, r2)
                else:
                    wait_out(g2 - RING, r2)
                    start_gather(g2, r2)
            else:
                @pl.when(g2 < G)
                def _():
                    wait_out(g2 - RING, r2)
                    start_gather(g2, r2)

        for g in range(AHEAD):
            start_gather(g, g)
        for r in range(RING):
            visit(r, r, first_lap=True)

        def step(s, _):
            for r in range(RING):
                visit(s * RING + r, r, first_lap=False)
            return 0

        lax.fori_loop(1, G // RING, step, 0)

        for r in range(RING):
            wait_out(G - RING + r, r)
        pltpu.sync_copy(mask_v, mask_hbm.at[wid])

    return emb_kernel


_emb_kernel = None


def kernel(x, W):
    global _emb_kernel
    if _emb_kernel is None:
        _emb_kernel = _make_kernel()
    xf = x.reshape(NW, G, CHUNK).astype(jnp.int32)
    emb, mask = _emb_kernel(xf, W)
    return emb.reshape(BATCH, SEQ, EMB), mask.reshape(BATCH, SEQ)
